# baseline (speedup 1.0000x reference)
import functools

import jax
import jax.numpy as jnp
import numpy as np
from jax.experimental import pallas as pl
from jax.experimental.pallas import tpu as pltpu
from jax.experimental.pallas import tpu_sc as plsc

N_CAT = 26
CAT_DIM = 101
EMB = 128
HID = 1024
NNUM = 13
B = 4096
INPUT_DIM = 2 * EMB + N_CAT * CAT_DIM + NNUM
XPAD = 2896
EPS = 1e-5

_NC = 2
_NS = 16
_NW = _NC * _NS
_BPW = B // _NW


def _x0_sc_kernel(user_emb, item_emb, cat_pad, user_ids, item_ids, cat_gidx):
    mesh = plsc.VectorSubcoreMesh(core_axis_name="c", subcore_axis_name="s")

    @functools.partial(
        pl.kernel,
        out_type=(
            jax.ShapeDtypeStruct((B, EMB), jnp.float32),
            jax.ShapeDtypeStruct((B, EMB), jnp.float32),
            jax.ShapeDtypeStruct((N_CAT, B, EMB), jnp.float32),
        ),
        mesh=mesh,
        scratch_types=[
            pltpu.VMEM((_BPW,), jnp.int32),
            pltpu.VMEM((_BPW, EMB), jnp.float32),
            pltpu.VMEM((_BPW, EMB), jnp.float32),
            pltpu.SemaphoreType.DMA,
        ],
    )
    def k(ue_hbm, ie_hbm, ce_hbm, uid_hbm, iid_hbm, cat_gidx_hbm,
          ue_out, ie_out, cat_out, idx_v, buf_emb, buf_cat, sem):
        wid = jax.lax.axis_index("s") * _NC + jax.lax.axis_index("c")
        base = wid * _BPW
        rows = pl.ds(base, _BPW)
        pltpu.sync_copy(uid_hbm.at[rows], idx_v)
        pltpu.async_copy(ue_hbm.at[idx_v], buf_emb, sem).wait()
        pltpu.sync_copy(buf_emb, ue_out.at[rows])
        pltpu.sync_copy(iid_hbm.at[rows], idx_v)
        pltpu.async_copy(ie_hbm.at[idx_v], buf_emb, sem).wait()
        pltpu.sync_copy(buf_emb, ie_out.at[rows])
        for i in range(N_CAT):
            pltpu.sync_copy(cat_gidx_hbm.at[i, rows], idx_v)
            pltpu.async_copy(ce_hbm.at[idx_v], buf_cat, sem).wait()
            pltpu.sync_copy(buf_cat, cat_out.at[i, rows])

    return k(user_emb, item_emb, cat_pad, user_ids, item_ids, cat_gidx)


def _bn(x, gamma, beta):
    mu = jnp.mean(x, axis=0)
    var = jnp.var(x, axis=0)
    return gamma * (x - mu) / jnp.sqrt(var + EPS) + beta


def kernel(user_ids, item_ids, cat_features, num_features, params):
    cat_pad = jnp.pad(params['cat_emb'],
                      ((0, 0), (0, 0), (0, EMB - CAT_DIM))).reshape(-1, EMB)
    cat_gidx = (cat_features.astype(jnp.int32)
                + (jnp.arange(N_CAT, dtype=jnp.int32) * 10000)[None, :]).T
    ue_g, ie_g, cat_g = _x0_sc_kernel(params['user_emb'], params['item_emb'],
                                      cat_pad, user_ids, item_ids, cat_gidx)
    cat_flat = cat_g[:, :, :CAT_DIM].transpose(1, 0, 2).reshape(
        B, N_CAT * CAT_DIM)
    x0 = jnp.concatenate([ue_g, ie_g, cat_flat, num_features], axis=1)

    deep = x0 @ params['W_init'] + params['b_init']
    for i in range(2):
        h = deep @ params[f'res{i}_W1'] + params[f'res{i}_b1']
        h = _bn(h, params[f'res{i}_g1'], params[f'res{i}_be1'])
        h = jax.nn.relu(h)
        h = h @ params[f'res{i}_W2'] + params[f'res{i}_b2']
        h = _bn(h, params[f'res{i}_g2'], params[f'res{i}_be2'])
        deep = jax.nn.relu(h + deep)
    cross = x0
    for i in range(3):
        s = cross @ params[f'cross{i}_w']
        cross = cross + cross * s + params[f'cross{i}_b']
    fin = jnp.concatenate([deep, cross], axis=1)
    out = fin @ params['W_final'] + params['b_final']
    return jnp.squeeze(out, axis=1)


# speedup vs baseline: 1.3380x; 1.3380x over previous
import functools

import jax
import jax.numpy as jnp
import numpy as np
from jax.experimental import pallas as pl
from jax.experimental.pallas import tpu as pltpu
from jax.experimental.pallas import tpu_sc as plsc

N_CAT = 26
CAT_CARD = 10000
CAT_DIM = 101
EMB = 128
HID = 1024
NNUM = 13
B = 4096
INPUT_DIM = 2 * EMB + N_CAT * CAT_DIM + NNUM
XW = (2 + N_CAT) * EMB
NQ = 4
WN = HID + NQ
EPS = 1e-5

_NC = 2
_NS = 16
_NW = _NC * _NS
_BPW = B // _NW

_BT = 256
_GRID = B // _BT


def _x0_sc_kernel(user_emb, item_emb, cat_pad, user_ids, item_ids, cat_gidx):
    mesh = plsc.VectorSubcoreMesh(core_axis_name="c", subcore_axis_name="s")

    @functools.partial(
        pl.kernel,
        out_type=jax.ShapeDtypeStruct((B, XW), jnp.float32),
        mesh=mesh,
        scratch_types=[
            pltpu.VMEM((_BPW,), jnp.int32),
            pltpu.VMEM((_BPW, EMB), jnp.float32),
            pltpu.VMEM((_BPW, EMB), jnp.float32),
            pltpu.SemaphoreType.DMA,
        ],
    )
    def k(ue_hbm, ie_hbm, ce_hbm, uid_hbm, iid_hbm, cat_gidx_hbm,
          x0_out, idx_v, buf_a, buf_b, sem):
        wid = jax.lax.axis_index("s") * _NC + jax.lax.axis_index("c")
        base = wid * _BPW
        rows = pl.ds(base, _BPW)
        pltpu.sync_copy(uid_hbm.at[rows], idx_v)
        pltpu.async_copy(ue_hbm.at[idx_v], buf_a, sem).wait()
        pltpu.sync_copy(buf_a, x0_out.at[rows, pl.ds(0, EMB)])
        pltpu.sync_copy(iid_hbm.at[rows], idx_v)
        pltpu.async_copy(ie_hbm.at[idx_v], buf_a, sem).wait()
        pltpu.sync_copy(buf_a, x0_out.at[rows, pl.ds(EMB, EMB)])
        for i in range(N_CAT):
            pltpu.sync_copy(cat_gidx_hbm.at[i, rows], idx_v)
            pltpu.async_copy(ce_hbm.at[idx_v], buf_b, sem).wait()
            pltpu.sync_copy(buf_b, x0_out.at[rows, pl.ds((2 + i) * EMB, EMB)])

    return k(user_emb, item_emb, cat_pad, user_ids, item_ids, cat_gidx)


def _bf(x):
    return x.astype(jnp.bfloat16)


def _bn_affine(acc_s, acc_sq, gamma, beta):
    mu = acc_s * (1.0 / B)
    var = acc_sq * (1.0 / B) - mu * mu
    a = gamma * jax.lax.rsqrt(var + EPS)
    return a, beta - a * mu


def _stage_a(x_ref, num_ref, wbig_ref, wnum_ref, b0_ref, w1_ref, b1_ref,
             g1_ref, be1_ref, cc_ref,
             deep0_ref, h1_ref, aff1_ref, cross_ref, acc_s, acc_sq):
    step = pl.program_id(0)

    @pl.when(step == 0)
    def _():
        acc_s[...] = jnp.zeros_like(acc_s)
        acc_sq[...] = jnp.zeros_like(acc_sq)

    acc = jnp.dot(_bf(x_ref[...]), wbig_ref[...],
                  preferred_element_type=jnp.float32)
    acc += jnp.dot(_bf(num_ref[...]), wnum_ref[...],
                   preferred_element_type=jnp.float32)
    deep0 = acc[:, :HID] + b0_ref[...]
    deep0_ref[...] = deep0
    q0 = acc[:, HID:HID + 1]
    q1 = acc[:, HID + 1:HID + 2]
    q2 = acc[:, HID + 2:HID + 3]
    q3 = acc[:, HID + 3:HID + 4]
    c01 = cc_ref[0, 0]
    c02 = cc_ref[0, 1]
    c12 = cc_ref[0, 2]
    c0f = cc_ref[0, 3]
    c1f = cc_ref[0, 4]
    c2f = cc_ref[0, 5]
    s0 = q0
    t0 = 1.0 + s0
    s1 = t0 * q1 + c01
    t1 = 1.0 + s1
    s2 = t0 * t1 * q2 + t1 * c02 + c12
    t2 = 1.0 + s2
    cross_ref[...] = t0 * t1 * t2 * q3 + t1 * t2 * c0f + t2 * c1f + c2f

    h1 = jnp.dot(_bf(deep0), w1_ref[...], preferred_element_type=jnp.float32)
    h1 += b1_ref[...]
    h1_ref[...] = h1
    acc_s[...] += jnp.sum(h1, axis=0, keepdims=True)
    acc_sq[...] += jnp.sum(h1 * h1, axis=0, keepdims=True)

    @pl.when(step == _GRID - 1)
    def _():
        a, s = _bn_affine(acc_s[...], acc_sq[...], g1_ref[...], be1_ref[...])
        aff1_ref[0:1, :] = a
        aff1_ref[1:2, :] = s


def _stage_mid(h_ref, aff_ref, w_ref, b_ref, g_ref, be_ref, skip_ref,
               hn_ref, affn_ref, skipn_ref, acc_s, acc_sq, *, has_skip,
               out_skip):
    step = pl.program_id(0)

    @pl.when(step == 0)
    def _():
        acc_s[...] = jnp.zeros_like(acc_s)
        acc_sq[...] = jnp.zeros_like(acc_sq)

    z = aff_ref[0:1, :] * h_ref[...] + aff_ref[1:2, :]
    if has_skip:
        z += skip_ref[...]
    z = jnp.maximum(z, 0.0)
    if out_skip:
        skipn_ref[...] = z
    hn = jnp.dot(_bf(z), w_ref[...], preferred_element_type=jnp.float32)
    hn += b_ref[...]
    hn_ref[...] = hn
    acc_s[...] += jnp.sum(hn, axis=0, keepdims=True)
    acc_sq[...] += jnp.sum(hn * hn, axis=0, keepdims=True)

    @pl.when(step == _GRID - 1)
    def _():
        a, s = _bn_affine(acc_s[...], acc_sq[...], g_ref[...], be_ref[...])
        affn_ref[0:1, :] = a
        affn_ref[1:2, :] = s


def _stage_e(h_ref, aff_ref, skip_ref, cross_ref, wf_ref, cc_ref, out_ref):
    z = aff_ref[0:1, :] * h_ref[...] + aff_ref[1:2, :] + skip_ref[...]
    z = jnp.maximum(z, 0.0)
    dot = jnp.sum(z * wf_ref[...], axis=1, keepdims=True)
    out_ref[...] = dot + cross_ref[...] + cc_ref[0, 0]


def _bspec(shape, resident=False):
    if resident:
        return pl.BlockSpec(shape, lambda b: tuple(0 for _ in shape))
    return pl.BlockSpec(shape, lambda b: (b,) + tuple(0 for _ in shape[1:]))


_F32 = jnp.float32


def _dense_tc(x0p, num, wbig, wnum, b0, w1a, b1a, g1a, be1a, w2a, b2a, g2a,
              be2a, w1b, b1b, g1b, be1b, w2b, b2b, g2b, be2b, wf, ccA, ccE):
    act = jax.ShapeDtypeStruct((B, HID), _F32)
    aff = jax.ShapeDtypeStruct((8, HID), _F32)

    deep0, h1, aff1, crossv = pl.pallas_call(
        _stage_a,
        grid=(_GRID,),
        in_specs=[
            _bspec((_BT, XW)),
            _bspec((_BT, NNUM)),
            _bspec((XW, WN), True),
            _bspec((NNUM, WN), True),
            _bspec((1, HID), True),
            _bspec((HID, HID), True),
            _bspec((1, HID), True),
            _bspec((1, HID), True),
            _bspec((1, HID), True),
            _bspec((1, 8), True),
        ],
        out_specs=[
            _bspec((_BT, HID)),
            _bspec((_BT, HID)),
            _bspec((8, HID), True),
            _bspec((_BT, 1)),
        ],
        out_shape=[act, act, aff, jax.ShapeDtypeStruct((B, 1), _F32)],
        scratch_shapes=[pltpu.VMEM((1, HID), _F32), pltpu.VMEM((1, HID), _F32)],
    )(x0p, num, wbig, wnum, b0, w1a, b1a, g1a, be1a, ccA)

    def mid(h, affin, w, b, g, be, skip, has_skip, out_skip):
        kern = functools.partial(_stage_mid, has_skip=has_skip,
                                 out_skip=out_skip)
        in_specs = [
            _bspec((_BT, HID)),
            _bspec((8, HID), True),
            _bspec((HID, HID), True),
            _bspec((1, HID), True),
            _bspec((1, HID), True),
            _bspec((1, HID), True),
            _bspec((_BT, HID)),
        ]
        out_specs = [_bspec((_BT, HID)), _bspec((8, HID), True),
                     _bspec((_BT, HID))]
        out_shape = [act, aff, act]
        return pl.pallas_call(
            kern,
            grid=(_GRID,),
            in_specs=in_specs,
            out_specs=out_specs,
            out_shape=out_shape,
            scratch_shapes=[pltpu.VMEM((1, HID), _F32),
                            pltpu.VMEM((1, HID), _F32)],
        )(h, affin, w, b, g, be, skip)

    h2, aff2, _ = mid(h1, aff1, w2a, b2a, g2a, be2a, deep0,
                      has_skip=False, out_skip=False)
    h3, aff3, deep1 = mid(h2, aff2, w1b, b1b, g1b, be1b, deep0,
                          has_skip=True, out_skip=True)
    h4, aff4, _ = mid(h3, aff3, w2b, b2b, g2b, be2b, deep1,
                      has_skip=False, out_skip=False)
    out = pl.pallas_call(
        _stage_e,
        grid=(_GRID,),
        in_specs=[
            _bspec((_BT, HID)),
            _bspec((8, HID), True),
            _bspec((_BT, HID)),
            _bspec((_BT, 1)),
            _bspec((1, HID), True),
            _bspec((1, 8), True),
        ],
        out_specs=[_bspec((_BT, 1))],
        out_shape=[jax.ShapeDtypeStruct((B, 1), _F32)],
    )(h4, aff4, deep1, crossv, wf, ccE)[0]
    return out


def kernel(user_ids, item_ids, cat_features, num_features, params):
    f32 = jnp.float32
    cat_pad = jnp.pad(params['cat_emb'],
                      ((0, 0), (0, 0), (0, EMB - CAT_DIM))).reshape(-1, EMB)
    cat_gidx = (cat_features.astype(jnp.int32)
                + (jnp.arange(N_CAT, dtype=jnp.int32) * CAT_CARD)[None, :]).T
    x0p = _x0_sc_kernel(params['user_emb'], params['item_emb'], cat_pad,
                        user_ids, item_ids, cat_gidx)

    w0 = params['cross0_w']
    w1 = params['cross1_w']
    w2 = params['cross2_w']
    wf_cross = params['W_final'][HID:]
    wf_deep = params['W_final'][:HID]
    wm = jnp.concatenate([params['W_init'], w0, w1, w2, wf_cross], axis=1)
    wm = wm.astype(jnp.bfloat16)
    w_ui = wm[:2 * EMB]
    w_cat = jnp.pad(wm[2 * EMB:2 * EMB + N_CAT * CAT_DIM]
                    .reshape(N_CAT, CAT_DIM, WN),
                    ((0, 0), (0, EMB - CAT_DIM), (0, 0))).reshape(-1, WN)
    wbig = jnp.concatenate([w_ui, w_cat], axis=0)
    wnum = wm[2 * EMB + N_CAT * CAT_DIM:]

    b0v = params['cross0_b']
    b1v = params['cross1_b']
    b2v = params['cross2_b']
    wfc = wf_cross[:, 0]
    w1v = w1[:, 0]
    w2v = w2[:, 0]
    ccA = jnp.stack([jnp.dot(b0v, w1v), jnp.dot(b0v, w2v), jnp.dot(b1v, w2v),
                     jnp.dot(b0v, wfc), jnp.dot(b1v, wfc), jnp.dot(b2v, wfc),
                     0.0, 0.0]).astype(f32)[None, :]
    ccE = jnp.concatenate([params['b_final'],
                           jnp.zeros((7,), f32)])[None, :]

    def row(v):
        return v.astype(f32)[None, :]

    out2d = _dense_tc(
        x0p, num_features, wbig, wnum, row(params['b_init']),
        params['res0_W1'].astype(jnp.bfloat16), row(params['res0_b1']),
        row(params['res0_g1']), row(params['res0_be1']),
        params['res0_W2'].astype(jnp.bfloat16), row(params['res0_b2']),
        row(params['res0_g2']), row(params['res0_be2']),
        params['res1_W1'].astype(jnp.bfloat16), row(params['res1_b1']),
        row(params['res1_g1']), row(params['res1_be1']),
        params['res1_W2'].astype(jnp.bfloat16), row(params['res1_b2']),
        row(params['res1_g2']), row(params['res1_be2']),
        row(wf_deep[:, 0]), ccA, ccE)
    return out2d.reshape(B)
